# baseline (speedup 1.0000x reference)
import functools
import math

import jax
import jax.numpy as jnp
from jax import lax
from jax.experimental import pallas as pl
from jax.experimental.pallas import tpu as pltpu
from jax.experimental.pallas import tpu_sc as plsc

D_M = 64
SCALE = math.sqrt(D_M)
LANES = 16

try:
    _info = plsc.get_sparse_core_info()
    NC, NS = _info.num_cores, _info.num_subcores
except Exception:
    NC, NS = 2, 16
NW = NC * NS

NBUF = 4
PF = 2


def _emb_body(C, n_chunks, b_per_w,
              x_hbm, table_hbm, out_hbm, idx_all, rows_v, sem_in, sem_out):
    wid = lax.axis_index("s") * NC + lax.axis_index("c")
    base = wid * b_per_w

    def gather(g, b):
        return pltpu.make_async_copy(
            table_hbm.at[idx_all.at[pl.ds(g * C, C)]], rows_v.at[b],
            sem_in.at[b])

    def write(g, b):
        return pltpu.make_async_copy(
            rows_v.at[b], out_hbm.at[pl.ds(base + g * C, C)], sem_out.at[b])

    pltpu.sync_copy(x_hbm.at[pl.ds(base, b_per_w)], idx_all)
    for b in range(PF):
        gather(b, b).start()

    @pl.loop(0, n_chunks, step=NBUF)
    def _outer(g0):
        for b in range(NBUF):
            g = g0 + b
            bp = (b + PF) % NBUF
            @pl.when(g + PF < n_chunks)
            def _pf():
                @pl.when(g + PF - NBUF >= 0)
                def _drain():
                    write(g + PF - NBUF, bp).wait()
                gather(g + PF, bp).start()

            gather(g, b).wait()
            write(g, b).start()

    for b in range(NBUF):
        write(n_chunks - NBUF + b, (n_chunks - NBUF + b) % NBUF).wait()


def _emb_lookup(xf, table):
    B = xf.shape[0]
    b_per_w = B // NW
    C = 400
    n_chunks = b_per_w // C

    mesh = plsc.VectorSubcoreMesh(core_axis_name="c", subcore_axis_name="s")
    body = functools.partial(_emb_body, C, n_chunks, b_per_w)
    return pl.kernel(
        body,
        out_type=jax.ShapeDtypeStruct((B, D_M), jnp.float32),
        mesh=mesh,
        compiler_params=pltpu.CompilerParams(use_tc_tiling_on_sc=False),
        scratch_types=[
            pltpu.VMEM((b_per_w,), jnp.int32),
            pltpu.VMEM((NBUF, C, D_M), jnp.float32),
            pltpu.SemaphoreType.DMA((NBUF,)),
            pltpu.SemaphoreType.DMA((NBUF,)),
        ],
    )(xf, table)


def kernel(x, table):
    B0, S = x.shape
    xf = x.reshape(B0 * S).astype(jnp.int32)
    out = _emb_lookup(xf, table)
    return out.reshape(B0, S, D_M) * SCALE


# speedup vs baseline: 1.2122x; 1.2122x over previous
import functools
import math

import jax
import jax.numpy as jnp
from jax import lax
from jax.experimental import pallas as pl
from jax.experimental.pallas import tpu as pltpu
from jax.experimental.pallas import tpu_sc as plsc

D_M = 64
SCALE = math.sqrt(D_M)
LANES = 16

try:
    _info = plsc.get_sparse_core_info()
    NC, NS = _info.num_cores, _info.num_subcores
except Exception:
    NC, NS = 2, 16
NW = NC * NS

NBUF = 4
PF = 2


def _emb_body(C, n_chunks, b_per_w,
              x_hbm, table_hbm, out_hbm, idx_all, rows_v, sem_in, sem_out):
    wid = lax.axis_index("s") * NC + lax.axis_index("c")
    base = wid * b_per_w

    def gather(g, b):
        return pltpu.make_async_copy(
            table_hbm.at[idx_all.at[pl.ds(g * C, C)]], rows_v.at[b],
            sem_in.at[b])

    def write(g, b):
        return pltpu.make_async_copy(
            rows_v.at[b], out_hbm.at[pl.ds(base + g * C, C)], sem_out.at[b])

    pltpu.sync_copy(x_hbm.at[pl.ds(base, b_per_w)], idx_all)
    for b in range(PF):
        gather(b, b).start()

    @pl.loop(0, n_chunks, step=NBUF)
    def _outer(g0):
        for b in range(NBUF):
            g = g0 + b
            bp = (b + PF) % NBUF
            @pl.when(g + PF < n_chunks)
            def _pf():
                @pl.when(g + PF - NBUF >= 0)
                def _drain():
                    write(g + PF - NBUF, bp).wait()
                gather(g + PF, bp).start()

            gather(g, b).wait()

            @pl.loop(0, C, unroll=8)
            def _srow(r):
                for j in range(D_M // LANES):
                    sl = (r, pl.ds(j * LANES, LANES))
                    rows_v[(b, *sl)] = rows_v[(b, *sl)] * SCALE

            write(g, b).start()

    for b in range(NBUF):
        write(n_chunks - NBUF + b, (n_chunks - NBUF + b) % NBUF).wait()


def _emb_lookup(xf, table):
    B = xf.shape[0]
    b_per_w = B // NW
    C = 400
    n_chunks = b_per_w // C

    mesh = plsc.VectorSubcoreMesh(core_axis_name="c", subcore_axis_name="s")
    body = functools.partial(_emb_body, C, n_chunks, b_per_w)
    return pl.kernel(
        body,
        out_type=jax.ShapeDtypeStruct((B, D_M), jnp.float32),
        mesh=mesh,
        compiler_params=pltpu.CompilerParams(use_tc_tiling_on_sc=False,
                                             skip_device_barrier=True),
        scratch_types=[
            pltpu.VMEM((b_per_w,), jnp.int32),
            pltpu.VMEM((NBUF, C, D_M), jnp.float32),
            pltpu.SemaphoreType.DMA((NBUF,)),
            pltpu.SemaphoreType.DMA((NBUF,)),
        ],
    )(xf, table)


def kernel(x, table):
    B0, S = x.shape
    xf = x.reshape(B0 * S).astype(jnp.int32)
    out = _emb_lookup(xf, table)
    return out.reshape(B0, S, D_M)
